# baseline (speedup 1.0000x reference)
import functools

import jax
import jax.numpy as jnp
from jax import lax
from jax.experimental import pallas as pl
from jax.experimental.pallas import tpu as pltpu
from jax.experimental.pallas import tpu_sc as plsc

N = 10000
E = 320000
D = 128
G = 64

NC = 2
NS = 16
NW = NC * NS

K = 128
EPT = E // NS
CH = 160
RES = 16
NSTAGE = CH // RES
EPT_PAD = CH * K
N_ACC = 10112
RPT = N_ACC // NS
DH = D // 2


@functools.partial(
    pl.kernel,
    out_type=jax.ShapeDtypeStruct((NC, N_ACC, DH), jnp.float32),
    mesh=plsc.VectorSubcoreMesh(core_axis_name="c", subcore_axis_name="s"),
    scratch_types=[
        pltpu.VMEM_SHARED((N_ACC, DH), jnp.float32),
        pltpu.VMEM_SHARED((N_ACC, DH), jnp.float32),
        pltpu.VMEM((RES, K), jnp.int32),
        pltpu.VMEM((RES, K), jnp.int32),
        pltpu.VMEM((K, DH), jnp.float32),
        pltpu.VMEM((K, DH), jnp.float32),
        pltpu.SemaphoreType.DMA,
        pltpu.SemaphoreType.DMA,
        pltpu.SemaphoreType.DMA,
        pltpu.SemaphoreType.DMA,
    ],
)
def _sc_agg(xsplit_hbm, srcs_hbm, dsts_hbm, out_hbm,
            x_sp, acc, src_v, dst_v, b0, b1,
            sem_g0, sem_g1, sem_s0, sem_s1):
    c = lax.axis_index("c")
    s = lax.axis_index("s")
    stripe = pl.ds(s * RPT, RPT)

    pltpu.sync_copy(xsplit_hbm.at[c].at[stripe], x_sp.at[stripe])
    pltpu.sync_copy(xsplit_hbm.at[c].at[stripe], acc.at[stripe])
    plsc.subcore_barrier()

    for q in range(NSTAGE):
        pltpu.sync_copy(srcs_hbm.at[s].at[pl.ds(q * RES, RES)], src_v)
        pltpu.sync_copy(dsts_hbm.at[s].at[pl.ds(q * RES, RES)], dst_v)

        def pair(t, carry):
            j = 2 * t
            g0 = pltpu.async_copy(x_sp.at[src_v.at[j]], b0, sem_g0)
            g1 = pltpu.async_copy(x_sp.at[src_v.at[j + 1]], b1, sem_g1)
            g0.wait()
            s0 = pltpu.async_copy(b0, acc.at[dst_v.at[j]], sem_s0,
                                  add=True)
            g1.wait()
            s1 = pltpu.async_copy(b1, acc.at[dst_v.at[j + 1]], sem_s1,
                                  add=True)
            s0.wait()
            s1.wait()
            return carry

        lax.fori_loop(0, RES // 2, pair, 0)
    plsc.subcore_barrier()

    pltpu.sync_copy(acc.at[stripe], out_hbm.at[c].at[stripe])


def _mlp_block(parts, w1, b1, gamma, beta, w2, b2):
    h = jnp.concatenate([parts[0, :N], parts[1, :N]], axis=1)
    y = jnp.dot(h, w1, preferred_element_type=jnp.float32) + b1
    mu = jnp.mean(y, axis=0, keepdims=True)
    var = jnp.mean(y * y, axis=0, keepdims=True) - mu * mu
    yn = gamma * (y - mu) * lax.rsqrt(var + 1e-5) + beta
    y2 = jnp.dot(jnp.maximum(yn, 0.0), w2,
                 preferred_element_type=jnp.float32) + b2
    return jnp.maximum(y2, 0.0)


def _tc_layer_body(p_ref, w1_ref, b1_ref, g_ref, be_ref, w2_ref,
                   b2_ref, out_ref):
    h = _mlp_block(p_ref[...], w1_ref[...], b1_ref[...],
                   g_ref[...], be_ref[...], w2_ref[...], b2_ref[...])
    hpad = jnp.concatenate(
        [h, jnp.zeros((N_ACC - N, D), jnp.float32)], axis=0)
    out_ref[0] = hpad[:, :DH]
    out_ref[1] = hpad[:, DH:]


def _tc_layer3_body(p_ref, w1_ref, b1_ref, g_ref, be_ref, w2_ref,
                    b2_ref, batch_ref, fw1_ref, fb1_ref, fw2_ref, fb2_ref,
                    out_ref):
    h = _mlp_block(p_ref[...], w1_ref[...], b1_ref[...], g_ref[...],
                   be_ref[...], w2_ref[...], b2_ref[...])
    onehot = (lax.broadcasted_iota(jnp.int32, (G, N), 0)
              == batch_ref[...]).astype(jnp.float32)
    pooled = jnp.dot(onehot, h, preferred_element_type=jnp.float32)
    gact = jnp.maximum(
        jnp.dot(pooled, fw1_ref[...], preferred_element_type=jnp.float32)
        + fb1_ref[...], 0.0)
    out_ref[...] = (jnp.dot(gact, fw2_ref[...],
                            preferred_element_type=jnp.float32)
                    + fb2_ref[...])


def _tc_layer(parts, layer):
    return pl.pallas_call(
        _tc_layer_body,
        out_shape=jax.ShapeDtypeStruct((NC, N_ACC, DH), jnp.float32),
    )(parts,
      layer["W1"], layer["b1"].reshape(1, -1),
      layer["gamma"].reshape(1, -1), layer["beta"].reshape(1, -1),
      layer["W2"], layer["b2"].reshape(1, -1))


def _tc_layer3(parts, layer, batch_i32, final):
    return pl.pallas_call(
        _tc_layer3_body,
        out_shape=jax.ShapeDtypeStruct((G, D), jnp.float32),
    )(parts,
      layer["W1"], layer["b1"].reshape(1, -1),
      layer["gamma"].reshape(1, -1), layer["beta"].reshape(1, -1),
      layer["W2"], layer["b2"].reshape(1, -1),
      batch_i32.reshape(1, -1),
      final["W1"], final["b1"].reshape(1, -1),
      final["W2"], final["b2"].reshape(1, -1))


def kernel(x, edge_index, batch, params):
    src = edge_index[0].astype(jnp.int32)
    dst = edge_index[1].astype(jnp.int32)
    pad = NS * EPT_PAD - E
    srcs = jnp.concatenate([src, jnp.zeros((pad,), jnp.int32)])
    dsts = jnp.concatenate([dst, jnp.full((pad,), N, jnp.int32)])
    srcs = srcs.reshape(NS, CH, K)
    dsts = dsts.reshape(NS, CH, K)
    batch_i32 = batch.astype(jnp.int32)

    xpad = jnp.concatenate(
        [x, jnp.zeros((N_ACC - N, D), jnp.float32)], axis=0)
    parts = jnp.stack([xpad[:, :DH], xpad[:, DH:]])

    for i, layer in enumerate(params["convs"]):
        parts = _sc_agg(parts, srcs, dsts)
        if i < len(params["convs"]) - 1:
            parts = _tc_layer(parts, layer)
        else:
            out = _tc_layer3(parts, layer, batch_i32, params["final"])
    return out


# speedup vs baseline: 1.0320x; 1.0320x over previous
import functools

import jax
import jax.numpy as jnp
from jax import lax
from jax.experimental import pallas as pl
from jax.experimental.pallas import tpu as pltpu
from jax.experimental.pallas import tpu_sc as plsc

N = 10000
E = 320000
D = 128
G = 64

NC = 2
NS = 16
NW = NC * NS

K = 128
EPT = E // NS
CH = 160
RES = 40
NSTAGE = CH // RES
EPT_PAD = CH * K
N_ACC = 10112
RPT = N_ACC // NS
DH = D // 2


@functools.partial(
    pl.kernel,
    out_type=jax.ShapeDtypeStruct((NC, N_ACC, DH), jnp.float32),
    mesh=plsc.VectorSubcoreMesh(core_axis_name="c", subcore_axis_name="s"),
    scratch_types=[
        pltpu.VMEM_SHARED((N_ACC, DH), jnp.float32),
        pltpu.VMEM_SHARED((N_ACC, DH), jnp.float32),
        pltpu.VMEM((RES, K), jnp.int32),
        pltpu.VMEM((RES, K), jnp.int32),
        pltpu.VMEM((K, DH), jnp.float32),
        pltpu.VMEM((K, DH), jnp.float32),
        pltpu.SemaphoreType.DMA,
        pltpu.SemaphoreType.DMA,
        pltpu.SemaphoreType.DMA,
        pltpu.SemaphoreType.DMA,
    ],
)
def _sc_agg(xsplit_hbm, srcs_hbm, dsts_hbm, out_hbm,
            x_sp, acc, src_v, dst_v, b0, b1,
            sem_g0, sem_g1, sem_s0, sem_s1):
    c = lax.axis_index("c")
    s = lax.axis_index("s")
    stripe = pl.ds(s * RPT, RPT)

    pltpu.sync_copy(xsplit_hbm.at[c].at[stripe], x_sp.at[stripe])
    pltpu.sync_copy(xsplit_hbm.at[c].at[stripe], acc.at[stripe])
    plsc.subcore_barrier()

    for q in range(NSTAGE):
        pltpu.sync_copy(srcs_hbm.at[s].at[pl.ds(q * RES, RES)], src_v)
        pltpu.sync_copy(dsts_hbm.at[s].at[pl.ds(q * RES, RES)], dst_v)

        def pair(t, carry):
            j = 2 * t
            g0 = pltpu.async_copy(x_sp.at[src_v.at[j]], b0, sem_g0)
            g1 = pltpu.async_copy(x_sp.at[src_v.at[j + 1]], b1, sem_g1)
            g0.wait()
            s0 = pltpu.async_copy(b0, acc.at[dst_v.at[j]], sem_s0,
                                  add=True)
            g1.wait()
            s1 = pltpu.async_copy(b1, acc.at[dst_v.at[j + 1]], sem_s1,
                                  add=True)
            s0.wait()
            s1.wait()
            return carry

        lax.fori_loop(0, RES // 2, pair, 0)
    plsc.subcore_barrier()

    pltpu.sync_copy(acc.at[stripe], out_hbm.at[c].at[stripe])


def _mlp_block(parts, w1, b1, gamma, beta, w2, b2):
    h = jnp.concatenate([parts[0, :N], parts[1, :N]], axis=1)
    y = jnp.dot(h, w1, preferred_element_type=jnp.float32) + b1
    mu = jnp.mean(y, axis=0, keepdims=True)
    var = jnp.mean(y * y, axis=0, keepdims=True) - mu * mu
    yn = gamma * (y - mu) * lax.rsqrt(var + 1e-5) + beta
    y2 = jnp.dot(jnp.maximum(yn, 0.0), w2,
                 preferred_element_type=jnp.float32) + b2
    return jnp.maximum(y2, 0.0)


def _tc_layer_body(p_ref, w1_ref, b1_ref, g_ref, be_ref, w2_ref,
                   b2_ref, out_ref):
    h = _mlp_block(p_ref[...], w1_ref[...], b1_ref[...],
                   g_ref[...], be_ref[...], w2_ref[...], b2_ref[...])
    hpad = jnp.concatenate(
        [h, jnp.zeros((N_ACC - N, D), jnp.float32)], axis=0)
    out_ref[0] = hpad[:, :DH]
    out_ref[1] = hpad[:, DH:]


def _tc_layer3_body(p_ref, w1_ref, b1_ref, g_ref, be_ref, w2_ref,
                    b2_ref, batch_ref, fw1_ref, fb1_ref, fw2_ref, fb2_ref,
                    out_ref):
    h = _mlp_block(p_ref[...], w1_ref[...], b1_ref[...], g_ref[...],
                   be_ref[...], w2_ref[...], b2_ref[...])
    onehot = (lax.broadcasted_iota(jnp.int32, (G, N), 0)
              == batch_ref[...]).astype(jnp.float32)
    pooled = jnp.dot(onehot, h, preferred_element_type=jnp.float32)
    gact = jnp.maximum(
        jnp.dot(pooled, fw1_ref[...], preferred_element_type=jnp.float32)
        + fb1_ref[...], 0.0)
    out_ref[...] = (jnp.dot(gact, fw2_ref[...],
                            preferred_element_type=jnp.float32)
                    + fb2_ref[...])


def _tc_layer(parts, layer):
    return pl.pallas_call(
        _tc_layer_body,
        out_shape=jax.ShapeDtypeStruct((NC, N_ACC, DH), jnp.float32),
    )(parts,
      layer["W1"], layer["b1"].reshape(1, -1),
      layer["gamma"].reshape(1, -1), layer["beta"].reshape(1, -1),
      layer["W2"], layer["b2"].reshape(1, -1))


def _tc_layer3(parts, layer, batch_i32, final):
    return pl.pallas_call(
        _tc_layer3_body,
        out_shape=jax.ShapeDtypeStruct((G, D), jnp.float32),
    )(parts,
      layer["W1"], layer["b1"].reshape(1, -1),
      layer["gamma"].reshape(1, -1), layer["beta"].reshape(1, -1),
      layer["W2"], layer["b2"].reshape(1, -1),
      batch_i32.reshape(1, -1),
      final["W1"], final["b1"].reshape(1, -1),
      final["W2"], final["b2"].reshape(1, -1))


def kernel(x, edge_index, batch, params):
    src = edge_index[0].astype(jnp.int32)
    dst = edge_index[1].astype(jnp.int32)
    pad = NS * EPT_PAD - E
    srcs = jnp.concatenate([src, jnp.zeros((pad,), jnp.int32)])
    dsts = jnp.concatenate([dst, jnp.full((pad,), N, jnp.int32)])
    srcs = srcs.reshape(NS, CH, K)
    dsts = dsts.reshape(NS, CH, K)
    batch_i32 = batch.astype(jnp.int32)

    xpad = jnp.concatenate(
        [x, jnp.zeros((N_ACC - N, D), jnp.float32)], axis=0)
    parts = jnp.stack([xpad[:, :DH], xpad[:, DH:]])

    for i, layer in enumerate(params["convs"]):
        parts = _sc_agg(parts, srcs, dsts)
        if i < len(params["convs"]) - 1:
            parts = _tc_layer(parts, layer)
        else:
            out = _tc_layer3(parts, layer, batch_i32, params["final"])
    return out
